# baseline (speedup 1.0000x reference)
import functools

import jax
import jax.numpy as jnp
from jax import lax
from jax.experimental import pallas as pl
from jax.experimental.pallas import tpu as pltpu
from jax.experimental.pallas import tpu_sc as plsc

NEG_SLOPE = 0.2
EPS_BN = 1e-5

_TC_PARAMS = pltpu.CompilerParams(vmem_limit_bytes=100 * 1024 * 1024)

NC = 2
NS = 16
L = 16
CHUNK = 128


def _stage_a_body(x_ref, wsat_ref, bsat_ref, wnei_ref, bnei_ref, wfus_ref,
                  bfus_ref, w1_ref, as1_ref, ad1_ref,
                  h_ref, z_ref, s_ref, d_ref):
    x = x_ref[...]
    sat_in = x[:, :wsat_ref.shape[0]]
    nei_in = x[:, wsat_ref.shape[0]:]
    sat = jnp.maximum(
        jnp.dot(sat_in, wsat_ref[...], preferred_element_type=jnp.float32)
        + bsat_ref[...][None, :], 0.0)
    nei = jnp.maximum(
        jnp.dot(nei_in, wnei_ref[...], preferred_element_type=jnp.float32)
        + bnei_ref[...][None, :], 0.0)
    hdim = sat.shape[1]
    logit = (jnp.dot(sat, wfus_ref[...][:hdim, :],
                     preferred_element_type=jnp.float32)
             + jnp.dot(nei, wfus_ref[...][hdim:, :],
                       preferred_element_type=jnp.float32)
             + bfus_ref[...][None, :])
    gate = 1.0 / (1.0 + jnp.exp(-logit))
    h = gate * sat + (1.0 - gate) * nei
    z = jnp.dot(h, w1_ref[...], preferred_element_type=jnp.float32)
    h_ref[...] = h
    z_ref[...] = z
    s_ref[...] = jnp.dot(z, as1_ref[...], preferred_element_type=jnp.float32)
    d_ref[...] = jnp.dot(z, ad1_ref[...], preferred_element_type=jnp.float32)


def _stage_b_body(num_ref, den_ref, res_ref, bc_ref, g_ref, be_ref, w2_ref,
                  as2_ref, ad2_ref, h2_ref, z2_ref, s2_ref, d2_ref):
    n = res_ref.shape[0]
    num = jnp.concatenate(
        [num_ref[0, 0][:n] + num_ref[0, 1][:n],
         num_ref[1, 0][:n] + num_ref[1, 1][:n]], axis=1)
    den = den_ref[0][:n, 0:1] + den_ref[1][:n, 0:1]
    o = num / (den + 1e-16) + bc_ref[...][None, :]
    mu = jnp.sum(o, axis=0, keepdims=True) / n
    var = jnp.sum(o * o, axis=0, keepdims=True) / n - mu * mu
    hb = g_ref[...][None, :] * (o - mu) * lax.rsqrt(var + EPS_BN) \
        + be_ref[...][None, :]
    h2 = jnp.maximum(hb, 0.0) + res_ref[...]
    z2 = jnp.dot(h2, w2_ref[...], preferred_element_type=jnp.float32)
    h2_ref[...] = h2
    z2_ref[...] = z2
    s2_ref[...] = jnp.dot(z2, as2_ref[...], preferred_element_type=jnp.float32)
    d2_ref[...] = jnp.dot(z2, ad2_ref[...], preferred_element_type=jnp.float32)


def _stage_c_body(num_ref, den_ref, res_ref, bc_ref, g_ref, be_ref,
                  wf1_ref, bf1_ref, wf2_ref, bf2_ref, out_ref):
    n = res_ref.shape[0]
    num = jnp.concatenate(
        [num_ref[0, 0][:n] + num_ref[0, 1][:n],
         num_ref[1, 0][:n] + num_ref[1, 1][:n]], axis=1)
    den = den_ref[0][:n, 0:1] + den_ref[1][:n, 0:1]
    o = num / (den + 1e-16) + bc_ref[...][None, :]
    mu = jnp.sum(o, axis=0, keepdims=True) / n
    var = jnp.sum(o * o, axis=0, keepdims=True) / n - mu * mu
    hb = g_ref[...][None, :] * (o - mu) * lax.rsqrt(var + EPS_BN) \
        + be_ref[...][None, :]
    h3 = jnp.maximum(hb, 0.0) + res_ref[...]
    f = jnp.maximum(
        jnp.dot(h3, wf1_ref[...], preferred_element_type=jnp.float32)
        + bf1_ref[...][None, :], 0.0)
    out_ref[...] = jnp.dot(f, wf2_ref[...],
                           preferred_element_type=jnp.float32) \
        + bf2_ref[...][None, :]


@functools.lru_cache(maxsize=None)
def _make_edge_kernel(n, hdim, e_tot, e_pad):
    hh = hdim // 2
    nw = NC * NS
    per_worker = e_pad // nw
    chunks = per_worker // CHUNK
    NBUF = 3
    assert chunks % NBUF == 0 and chunks >= 2 * NBUF
    groups = chunks // NBUF
    n_pad = ((n + NS * 128 - 1) // (NS * 128)) * (NS * 128)
    rows_per_tile = n_pad // NS
    zb_rows = 128
    zcopies = rows_per_tile // zb_rows
    assert rows_per_tile % zb_rows == 0 and rows_per_tile % 8 == 0
    assert zb_rows <= CHUNK
    UNROLL = 16
    assert CHUNK % (UNROLL * L) == 0 or CHUNK % UNROLL == 0

    mesh = plsc.VectorSubcoreMesh(core_axis_name="c", subcore_axis_name="s",
                                  num_cores=NC, num_subcores=NS)

    @functools.partial(
        pl.kernel,
        mesh=mesh,
        compiler_params=pltpu.CompilerParams(needs_layout_passes=False,
                                             use_tc_tiling_on_sc=False),
        out_type=[jax.ShapeDtypeStruct((2, NC, n_pad, hh), jnp.float32),
                  jax.ShapeDtypeStruct((NC, n_pad, L), jnp.float32)],
        scratch_types=[
            pltpu.VMEM((n + L,), jnp.float32),
            pltpu.VMEM((n + L,), jnp.float32),
            pltpu.VMEM((chunks, CHUNK), jnp.int32),
            pltpu.VMEM((chunks, CHUNK), jnp.int32),
            pltpu.VMEM((NBUF, CHUNK), jnp.float32),
            pltpu.VMEM((NBUF, CHUNK, L), jnp.float32),
            pltpu.VMEM((NBUF, CHUNK, hh), jnp.float32),
            pltpu.VMEM_SHARED((n_pad, hh), jnp.float32),
            pltpu.VMEM_SHARED((n_pad, L), jnp.float32),
            pltpu.SemaphoreType.DMA((NBUF,)),
            pltpu.SemaphoreType.DMA((NBUF,)),
            pltpu.SemaphoreType.DMA((NBUF,)),
        ],
    )
    def edge_kernel(zlo_hbm, zhi_hbm, zs_hbm, zd_hbm, src_hbm, dst_hbm,
                    num_hbm, den_hbm,
                    zs_v, zd_v, srcall_v, dstall_v, wcur_v, wrow_v, zrows_v,
                    num_sh, den_sh, sem_g, sem_sn, sem_sd):
        cid = lax.axis_index("c")
        sid = lax.axis_index("s")
        wid = sid * NC + cid
        rbase = sid * rows_per_tile
        zeros16 = jnp.zeros((L,), jnp.float32)

        def zero_accs(zero_den):
            def zero_row(i, _):
                for j in range(hh // L):
                    zrows_v[0, i, pl.ds(j * L, L)] = zeros16
                wrow_v[0, i, :] = zeros16
                return 0

            lax.fori_loop(0, zb_rows, zero_row, 0)
            for i in range(zcopies):
                pltpu.sync_copy(
                    zrows_v.at[0, pl.ds(0, zb_rows)],
                    num_sh.at[pl.ds(rbase + i * zb_rows, zb_rows)])
                if zero_den:
                    pltpu.sync_copy(
                        wrow_v.at[0, pl.ds(0, zb_rows)],
                        den_sh.at[pl.ds(rbase + i * zb_rows, zb_rows)])

        zero_accs(True)

        pltpu.sync_copy(zs_hbm, zs_v.at[pl.ds(0, n)])
        pltpu.sync_copy(zd_hbm, zd_v.at[pl.ds(0, n)])
        zs_v[pl.ds(n, L)] = jnp.full((L,), -1e9, jnp.float32)
        zd_v[pl.ds(n, L)] = jnp.full((L,), -1e9, jnp.float32)
        pltpu.sync_copy(src_hbm.at[wid], srcall_v)
        pltpu.sync_copy(dst_hbm.at[wid], dstall_v)
        plsc.subcore_barrier()

        def gather_wait(b, z_hbm):
            pltpu.make_async_copy(
                z_hbm.at[srcall_v.at[0]], zrows_v.at[b], sem_g.at[b]).wait()

        def scatter_waits(b, den):
            pltpu.make_async_copy(
                zrows_v.at[b], num_sh.at[dstall_v.at[0]], sem_sn.at[b]).wait()
            if den:
                pltpu.make_async_copy(
                    wrow_v.at[b], den_sh.at[dstall_v.at[0]],
                    sem_sd.at[b]).wait()

        def compute_w(b, ci):
            @plsc.parallel_loop(0, CHUNK // L, step=1, unroll=8)
            def _(j):
                sv = srcall_v[ci, pl.ds(j * L, L)]
                dv = dstall_v[ci, pl.ds(j * L, L)]
                t = plsc.load_gather(zs_v, [sv]) + plsc.load_gather(zd_v, [dv])
                t = jnp.where(t > 0.0, t, t * NEG_SLOPE)
                wcur_v[b, pl.ds(j * L, L)] = jnp.exp(t)

        def scale_rows(b, p0):
            @plsc.parallel_loop(0, CHUNK, step=1, unroll=UNROLL)
            def _(e):
                wsp = plsc.load_gather(
                    wcur_v.at[b], [jnp.zeros((L,), jnp.int32) + e])
                if p0:
                    wrow_v[b, e, :] = wsp
                for j in range(hh // L):
                    zrows_v[b, e, pl.ds(j * L, L)] = \
                        zrows_v[b, e, pl.ds(j * L, L)] * wsp

        def run_pass(z_hbm, p0):
            for b in range(NBUF - 1):
                pltpu.async_copy(z_hbm.at[srcall_v.at[b]], zrows_v.at[b],
                                 sem_g.at[b])

            def group(g, _):
                for b in range(NBUF):
                    ci = g * NBUF + b
                    compute_w(b, ci)
                    gather_wait(b, z_hbm)
                    scale_rows(b, p0)
                    pltpu.async_copy(zrows_v.at[b],
                                     num_sh.at[dstall_v.at[ci]],
                                     sem_sn.at[b], add=True)
                    if p0:
                        pltpu.async_copy(wrow_v.at[b],
                                         den_sh.at[dstall_v.at[ci]],
                                         sem_sd.at[b], add=True)
                    bn = (b + NBUF - 1) % NBUF
                    nxt = ci + NBUF - 1

                    def prep():
                        scatter_waits(bn, p0)
                        pltpu.async_copy(z_hbm.at[srcall_v.at[nxt]],
                                         zrows_v.at[bn], sem_g.at[bn])

                    if b == 0:
                        @pl.when(g == 0)
                        def _():
                            pltpu.async_copy(z_hbm.at[srcall_v.at[nxt]],
                                             zrows_v.at[bn], sem_g.at[bn])

                        @pl.when(g > 0)
                        def _():
                            prep()
                    else:
                        @pl.when(nxt < chunks)
                        def _():
                            prep()
                return 0

            lax.fori_loop(0, groups, group, 0)
            for b in range(NBUF):
                scatter_waits(b, p0)

        run_pass(zlo_hbm, True)
        plsc.subcore_barrier()
        pltpu.sync_copy(num_sh.at[pl.ds(rbase, rows_per_tile)],
                        num_hbm.at[0, cid, pl.ds(rbase, rows_per_tile)])
        pltpu.sync_copy(den_sh.at[pl.ds(rbase, rows_per_tile)],
                        den_hbm.at[cid, pl.ds(rbase, rows_per_tile)])
        zero_accs(False)
        plsc.subcore_barrier()

        run_pass(zhi_hbm, False)
        plsc.subcore_barrier()
        pltpu.sync_copy(num_sh.at[pl.ds(rbase, rows_per_tile)],
                        num_hbm.at[1, cid, pl.ds(rbase, rows_per_tile)])

    return edge_kernel


def kernel(x, W_sat, b_sat, W_nei, b_nei, W_fus, b_fus, W1, a_src1, a_dst1,
           bc1, g1, be1, W2, a_src2, a_dst2, bc2, g2, be2, Wf1, bf1, Wf2,
           bf2, edge_index):
    n, _ = x.shape
    hdim = W1.shape[0]
    out_dim = Wf2.shape[1]
    e_edges = edge_index.shape[1]
    e_tot = e_edges + n
    nw = NC * NS
    e_pad = ((e_tot + nw * CHUNK - 1) // (nw * CHUNK)) * (nw * CHUNK)

    chunks = e_pad // (nw * CHUNK)
    si = jnp.arange(n, dtype=edge_index.dtype)
    pad = jnp.full((e_pad - e_tot,), n, edge_index.dtype)
    src = jnp.concatenate([edge_index[0], si, pad]).reshape(nw, chunks, CHUNK)
    dst = jnp.concatenate([edge_index[1], si, pad]).reshape(nw, chunks, CHUNK)
    zrow_pad = jnp.zeros((16, hdim // 2), jnp.float32)

    f32 = jnp.float32
    stage_a = pl.pallas_call(
        _stage_a_body,
        compiler_params=_TC_PARAMS,
        out_shape=[jax.ShapeDtypeStruct((n, hdim), f32),
                   jax.ShapeDtypeStruct((n, hdim), f32),
                   jax.ShapeDtypeStruct((n, 1), f32),
                   jax.ShapeDtypeStruct((n, 1), f32)],
    )
    h, z1, s1, d1 = stage_a(x, W_sat, b_sat, W_nei, b_nei, W_fus, b_fus, W1,
                            a_src1.reshape(hdim, 1), a_dst1.reshape(hdim, 1))

    hh = hdim // 2
    edge_fn = _make_edge_kernel(n, hdim, e_tot, e_pad)
    num1, den1 = edge_fn(
        jnp.concatenate([z1[:, :hh], zrow_pad], axis=0),
        jnp.concatenate([z1[:, hh:], zrow_pad], axis=0),
        s1.reshape(n), d1.reshape(n), src, dst)

    stage_b = pl.pallas_call(
        _stage_b_body,
        compiler_params=_TC_PARAMS,
        out_shape=[jax.ShapeDtypeStruct((n, hdim), f32),
                   jax.ShapeDtypeStruct((n, hdim), f32),
                   jax.ShapeDtypeStruct((n, 1), f32),
                   jax.ShapeDtypeStruct((n, 1), f32)],
    )
    h2, z2, s2, d2 = stage_b(num1, den1, h, bc1, g1, be1, W2,
                             a_src2.reshape(hdim, 1), a_dst2.reshape(hdim, 1))

    num2, den2 = edge_fn(
        jnp.concatenate([z2[:, :hh], zrow_pad], axis=0),
        jnp.concatenate([z2[:, hh:], zrow_pad], axis=0),
        s2.reshape(n), d2.reshape(n), src, dst)

    stage_c = pl.pallas_call(
        _stage_c_body,
        compiler_params=_TC_PARAMS,
        out_shape=jax.ShapeDtypeStruct((n, out_dim), f32),
    )
    return stage_c(num2, den2, h2, bc2, g2, be2, Wf1, bf1, Wf2, bf2)


# speedup vs baseline: 1.0096x; 1.0096x over previous
import functools

import jax
import jax.numpy as jnp
from jax import lax
from jax.experimental import pallas as pl
from jax.experimental.pallas import tpu as pltpu
from jax.experimental.pallas import tpu_sc as plsc

NEG_SLOPE = 0.2
EPS_BN = 1e-5

_TC_PARAMS = pltpu.CompilerParams(vmem_limit_bytes=100 * 1024 * 1024)

NC = 2
NS = 16
L = 16
CHUNK = 128


def _stage_a_body(x_ref, wsat_ref, bsat_ref, wnei_ref, bnei_ref, wfus_ref,
                  bfus_ref, w1_ref, as1_ref, ad1_ref,
                  h_ref, z_ref, s_ref, d_ref):
    x = x_ref[...]
    sat_in = x[:, :wsat_ref.shape[0]]
    nei_in = x[:, wsat_ref.shape[0]:]
    sat = jnp.maximum(
        jnp.dot(sat_in, wsat_ref[...], preferred_element_type=jnp.float32)
        + bsat_ref[...][None, :], 0.0)
    nei = jnp.maximum(
        jnp.dot(nei_in, wnei_ref[...], preferred_element_type=jnp.float32)
        + bnei_ref[...][None, :], 0.0)
    hdim = sat.shape[1]
    logit = (jnp.dot(sat, wfus_ref[...][:hdim, :],
                     preferred_element_type=jnp.float32)
             + jnp.dot(nei, wfus_ref[...][hdim:, :],
                       preferred_element_type=jnp.float32)
             + bfus_ref[...][None, :])
    gate = 1.0 / (1.0 + jnp.exp(-logit))
    h = gate * sat + (1.0 - gate) * nei
    z = jnp.dot(h, w1_ref[...], preferred_element_type=jnp.float32)
    h_ref[...] = h
    z_ref[...] = z
    s_ref[...] = jnp.dot(z, as1_ref[...], preferred_element_type=jnp.float32)
    d_ref[...] = jnp.dot(z, ad1_ref[...], preferred_element_type=jnp.float32)


def _stage_b_body(num_ref, den_ref, res_ref, bc_ref, g_ref, be_ref, w2_ref,
                  as2_ref, ad2_ref, h2_ref, z2_ref, s2_ref, d2_ref):
    n = res_ref.shape[0]
    num = jnp.concatenate(
        [num_ref[0, 0][:n] + num_ref[0, 1][:n],
         num_ref[1, 0][:n] + num_ref[1, 1][:n]], axis=1)
    den = den_ref[0][:n, 0:1] + den_ref[1][:n, 0:1]
    o = num / (den + 1e-16) + bc_ref[...][None, :]
    mu = jnp.sum(o, axis=0, keepdims=True) / n
    var = jnp.sum(o * o, axis=0, keepdims=True) / n - mu * mu
    hb = g_ref[...][None, :] * (o - mu) * lax.rsqrt(var + EPS_BN) \
        + be_ref[...][None, :]
    h2 = jnp.maximum(hb, 0.0) + res_ref[...]
    z2 = jnp.dot(h2, w2_ref[...], preferred_element_type=jnp.float32)
    h2_ref[...] = h2
    z2_ref[...] = z2
    s2_ref[...] = jnp.dot(z2, as2_ref[...], preferred_element_type=jnp.float32)
    d2_ref[...] = jnp.dot(z2, ad2_ref[...], preferred_element_type=jnp.float32)


def _stage_c_body(num_ref, den_ref, res_ref, bc_ref, g_ref, be_ref,
                  wf1_ref, bf1_ref, wf2_ref, bf2_ref, out_ref):
    n = res_ref.shape[0]
    num = jnp.concatenate(
        [num_ref[0, 0][:n] + num_ref[0, 1][:n],
         num_ref[1, 0][:n] + num_ref[1, 1][:n]], axis=1)
    den = den_ref[0][:n, 0:1] + den_ref[1][:n, 0:1]
    o = num / (den + 1e-16) + bc_ref[...][None, :]
    mu = jnp.sum(o, axis=0, keepdims=True) / n
    var = jnp.sum(o * o, axis=0, keepdims=True) / n - mu * mu
    hb = g_ref[...][None, :] * (o - mu) * lax.rsqrt(var + EPS_BN) \
        + be_ref[...][None, :]
    h3 = jnp.maximum(hb, 0.0) + res_ref[...]
    f = jnp.maximum(
        jnp.dot(h3, wf1_ref[...], preferred_element_type=jnp.float32)
        + bf1_ref[...][None, :], 0.0)
    out_ref[...] = jnp.dot(f, wf2_ref[...],
                           preferred_element_type=jnp.float32) \
        + bf2_ref[...][None, :]


@functools.lru_cache(maxsize=None)
def _make_edge_kernel(n, hdim, e_tot, e_pad):
    hh = hdim // 2
    nw = NC * NS
    per_worker = e_pad // nw
    chunks = per_worker // CHUNK
    NBUF = 3
    assert chunks % NBUF == 0 and chunks >= 2 * NBUF
    groups = chunks // NBUF
    n_pad = ((n + NS * 128 - 1) // (NS * 128)) * (NS * 128)
    rows_per_tile = n_pad // NS
    zb_rows = 128
    zcopies = rows_per_tile // zb_rows
    assert rows_per_tile % zb_rows == 0 and rows_per_tile % 8 == 0
    assert zb_rows <= CHUNK
    UNROLL = 8
    assert CHUNK % (UNROLL * L) == 0 or CHUNK % UNROLL == 0

    mesh = plsc.VectorSubcoreMesh(core_axis_name="c", subcore_axis_name="s",
                                  num_cores=NC, num_subcores=NS)

    @functools.partial(
        pl.kernel,
        mesh=mesh,
        compiler_params=pltpu.CompilerParams(needs_layout_passes=False,
                                             use_tc_tiling_on_sc=False),
        out_type=[jax.ShapeDtypeStruct((2, NC, n_pad, hh), jnp.float32),
                  jax.ShapeDtypeStruct((NC, n_pad, L), jnp.float32)],
        scratch_types=[
            pltpu.VMEM((n + L,), jnp.float32),
            pltpu.VMEM((n + L,), jnp.float32),
            pltpu.VMEM((chunks, CHUNK), jnp.int32),
            pltpu.VMEM((chunks, CHUNK), jnp.int32),
            pltpu.VMEM((NBUF, CHUNK), jnp.float32),
            pltpu.VMEM((NBUF, CHUNK, L), jnp.float32),
            pltpu.VMEM((NBUF, CHUNK, hh), jnp.float32),
            pltpu.VMEM_SHARED((n_pad, hh), jnp.float32),
            pltpu.VMEM_SHARED((n_pad, L), jnp.float32),
            pltpu.SemaphoreType.DMA((NBUF,)),
            pltpu.SemaphoreType.DMA((NBUF,)),
            pltpu.SemaphoreType.DMA((NBUF,)),
        ],
    )
    def edge_kernel(zlo_hbm, zhi_hbm, zs_hbm, zd_hbm, src_hbm, dst_hbm,
                    num_hbm, den_hbm,
                    zs_v, zd_v, srcall_v, dstall_v, wcur_v, wrow_v, zrows_v,
                    num_sh, den_sh, sem_g, sem_sn, sem_sd):
        cid = lax.axis_index("c")
        sid = lax.axis_index("s")
        wid = sid * NC + cid
        rbase = sid * rows_per_tile
        zeros16 = jnp.zeros((L,), jnp.float32)

        def zero_accs(zero_den):
            def zero_row(i, _):
                for j in range(hh // L):
                    zrows_v[0, i, pl.ds(j * L, L)] = zeros16
                wrow_v[0, i, :] = zeros16
                return 0

            lax.fori_loop(0, zb_rows, zero_row, 0)
            for i in range(zcopies):
                pltpu.sync_copy(
                    zrows_v.at[0, pl.ds(0, zb_rows)],
                    num_sh.at[pl.ds(rbase + i * zb_rows, zb_rows)])
                if zero_den:
                    pltpu.sync_copy(
                        wrow_v.at[0, pl.ds(0, zb_rows)],
                        den_sh.at[pl.ds(rbase + i * zb_rows, zb_rows)])

        zero_accs(True)

        pltpu.sync_copy(zs_hbm, zs_v.at[pl.ds(0, n)])
        pltpu.sync_copy(zd_hbm, zd_v.at[pl.ds(0, n)])
        zs_v[pl.ds(n, L)] = jnp.full((L,), -1e9, jnp.float32)
        zd_v[pl.ds(n, L)] = jnp.full((L,), -1e9, jnp.float32)
        pltpu.sync_copy(src_hbm.at[wid], srcall_v)
        pltpu.sync_copy(dst_hbm.at[wid], dstall_v)
        plsc.subcore_barrier()

        def gather_wait(b, z_hbm):
            pltpu.make_async_copy(
                z_hbm.at[srcall_v.at[0]], zrows_v.at[b], sem_g.at[b]).wait()

        def scatter_waits(b, den):
            pltpu.make_async_copy(
                zrows_v.at[b], num_sh.at[dstall_v.at[0]], sem_sn.at[b]).wait()
            if den:
                pltpu.make_async_copy(
                    wrow_v.at[b], den_sh.at[dstall_v.at[0]],
                    sem_sd.at[b]).wait()

        def compute_w(b, ci):
            @plsc.parallel_loop(0, CHUNK // L, step=1, unroll=4)
            def _(j):
                sv = srcall_v[ci, pl.ds(j * L, L)]
                dv = dstall_v[ci, pl.ds(j * L, L)]
                t = plsc.load_gather(zs_v, [sv]) + plsc.load_gather(zd_v, [dv])
                t = jnp.where(t > 0.0, t, t * NEG_SLOPE)
                wcur_v[b, pl.ds(j * L, L)] = jnp.exp(t)

        def scale_rows(b, p0):
            @plsc.parallel_loop(0, CHUNK, step=1, unroll=UNROLL)
            def _(e):
                wsp = plsc.load_gather(
                    wcur_v.at[b], [jnp.zeros((L,), jnp.int32) + e])
                if p0:
                    wrow_v[b, e, :] = wsp
                for j in range(hh // L):
                    zrows_v[b, e, pl.ds(j * L, L)] = \
                        zrows_v[b, e, pl.ds(j * L, L)] * wsp

        def run_pass(z_hbm, p0):
            for b in range(NBUF - 1):
                pltpu.async_copy(z_hbm.at[srcall_v.at[b]], zrows_v.at[b],
                                 sem_g.at[b])

            def group(g, _):
                for b in range(NBUF):
                    ci = g * NBUF + b
                    compute_w(b, ci)
                    gather_wait(b, z_hbm)
                    scale_rows(b, p0)
                    pltpu.async_copy(zrows_v.at[b],
                                     num_sh.at[dstall_v.at[ci]],
                                     sem_sn.at[b], add=True)
                    if p0:
                        pltpu.async_copy(wrow_v.at[b],
                                         den_sh.at[dstall_v.at[ci]],
                                         sem_sd.at[b], add=True)
                    bn = (b + NBUF - 1) % NBUF
                    nxt = ci + NBUF - 1

                    def prep():
                        scatter_waits(bn, p0)
                        pltpu.async_copy(z_hbm.at[srcall_v.at[nxt]],
                                         zrows_v.at[bn], sem_g.at[bn])

                    if b == 0:
                        @pl.when(g == 0)
                        def _():
                            pltpu.async_copy(z_hbm.at[srcall_v.at[nxt]],
                                             zrows_v.at[bn], sem_g.at[bn])

                        @pl.when(g > 0)
                        def _():
                            prep()
                    else:
                        @pl.when(nxt < chunks)
                        def _():
                            prep()
                return 0

            lax.fori_loop(0, groups, group, 0)
            for b in range(NBUF):
                scatter_waits(b, p0)

        run_pass(zlo_hbm, True)
        plsc.subcore_barrier()
        pltpu.sync_copy(num_sh.at[pl.ds(rbase, rows_per_tile)],
                        num_hbm.at[0, cid, pl.ds(rbase, rows_per_tile)])
        pltpu.sync_copy(den_sh.at[pl.ds(rbase, rows_per_tile)],
                        den_hbm.at[cid, pl.ds(rbase, rows_per_tile)])
        zero_accs(False)
        plsc.subcore_barrier()

        run_pass(zhi_hbm, False)
        plsc.subcore_barrier()
        pltpu.sync_copy(num_sh.at[pl.ds(rbase, rows_per_tile)],
                        num_hbm.at[1, cid, pl.ds(rbase, rows_per_tile)])

    return edge_kernel


def kernel(x, W_sat, b_sat, W_nei, b_nei, W_fus, b_fus, W1, a_src1, a_dst1,
           bc1, g1, be1, W2, a_src2, a_dst2, bc2, g2, be2, Wf1, bf1, Wf2,
           bf2, edge_index):
    n, _ = x.shape
    hdim = W1.shape[0]
    out_dim = Wf2.shape[1]
    e_edges = edge_index.shape[1]
    e_tot = e_edges + n
    nw = NC * NS
    e_pad = ((e_tot + nw * CHUNK - 1) // (nw * CHUNK)) * (nw * CHUNK)

    chunks = e_pad // (nw * CHUNK)
    si = jnp.arange(n, dtype=edge_index.dtype)
    pad = jnp.full((e_pad - e_tot,), n, edge_index.dtype)
    src = jnp.concatenate([edge_index[0], si, pad]).reshape(nw, chunks, CHUNK)
    dst = jnp.concatenate([edge_index[1], si, pad]).reshape(nw, chunks, CHUNK)
    zrow_pad = jnp.zeros((16, hdim // 2), jnp.float32)

    f32 = jnp.float32
    stage_a = pl.pallas_call(
        _stage_a_body,
        compiler_params=_TC_PARAMS,
        out_shape=[jax.ShapeDtypeStruct((n, hdim), f32),
                   jax.ShapeDtypeStruct((n, hdim), f32),
                   jax.ShapeDtypeStruct((n, 1), f32),
                   jax.ShapeDtypeStruct((n, 1), f32)],
    )
    h, z1, s1, d1 = stage_a(x, W_sat, b_sat, W_nei, b_nei, W_fus, b_fus, W1,
                            a_src1.reshape(hdim, 1), a_dst1.reshape(hdim, 1))

    hh = hdim // 2
    edge_fn = _make_edge_kernel(n, hdim, e_tot, e_pad)
    num1, den1 = edge_fn(
        jnp.concatenate([z1[:, :hh], zrow_pad], axis=0),
        jnp.concatenate([z1[:, hh:], zrow_pad], axis=0),
        s1.reshape(n), d1.reshape(n), src, dst)

    stage_b = pl.pallas_call(
        _stage_b_body,
        compiler_params=_TC_PARAMS,
        out_shape=[jax.ShapeDtypeStruct((n, hdim), f32),
                   jax.ShapeDtypeStruct((n, hdim), f32),
                   jax.ShapeDtypeStruct((n, 1), f32),
                   jax.ShapeDtypeStruct((n, 1), f32)],
    )
    h2, z2, s2, d2 = stage_b(num1, den1, h, bc1, g1, be1, W2,
                             a_src2.reshape(hdim, 1), a_dst2.reshape(hdim, 1))

    num2, den2 = edge_fn(
        jnp.concatenate([z2[:, :hh], zrow_pad], axis=0),
        jnp.concatenate([z2[:, hh:], zrow_pad], axis=0),
        s2.reshape(n), d2.reshape(n), src, dst)

    stage_c = pl.pallas_call(
        _stage_c_body,
        compiler_params=_TC_PARAMS,
        out_shape=jax.ShapeDtypeStruct((n, out_dim), f32),
    )
    return stage_c(num2, den2, h2, bc2, g2, be2, Wf1, bf1, Wf2, bf2)
